# baseline (speedup 1.0000x reference)
import functools

import jax
import jax.numpy as jnp
from jax import lax
from jax.experimental import pallas as pl
from jax.experimental.pallas import tpu as pltpu
from jax.experimental.pallas import tpu_sc as plsc

NC = 2
NS = 16
LANES = 16
EB = 112


def _matmul_body(x_ref, w_ref, o_ref):
    o_ref[...] = jnp.dot(x_ref[...], w_ref[...],
                         preferred_element_type=jnp.float32)


def _matmul(x, W, block_rows=1000):
    n, d = x.shape
    o = W.shape[1]
    grid = n // block_rows
    return pl.pallas_call(
        _matmul_body,
        grid=(grid,),
        in_specs=[
            pl.BlockSpec((block_rows, d), lambda i: (i, 0)),
            pl.BlockSpec((d, o), lambda i: (0, 0)),
        ],
        out_specs=pl.BlockSpec((block_rows, o), lambda i: (i, 0)),
        out_shape=jax.ShapeDtypeStruct((n, o), jnp.float32),
    )(x, W)


def _combine_body(a_ref, b_ref, o_ref):
    o_ref[...] = jnp.maximum(a_ref[...] + b_ref[...], 0.0)


def _combine(a, b, block_rows=1000):
    n, o = a.shape
    grid = n // block_rows
    return pl.pallas_call(
        _combine_body,
        grid=(grid,),
        in_specs=[
            pl.BlockSpec((block_rows, o), lambda i: (i, 0)),
            pl.BlockSpec((block_rows, o), lambda i: (i, 0)),
        ],
        out_specs=pl.BlockSpec((block_rows, o), lambda i: (i, 0)),
        out_shape=jax.ShapeDtypeStruct((n, o), jnp.float32),
    )(a, b)


def _sc_aggregate(h, eib, whb, k0, k1, n, o):
    mesh = plsc.VectorSubcoreMesh(core_axis_name="c", subcore_axis_name="s")
    rows_per_tile = n // NS

    NB = 3
    NI = 4
    NW = 3

    @functools.partial(
        pl.kernel,
        out_type=jax.ShapeDtypeStruct((NC, n, o), jnp.float32),
        mesh=mesh,
        scratch_types=dict(
            idx_v=pltpu.VMEM((NI, 2, EB), jnp.int32),
            w_v=pltpu.VMEM((NW, EB), jnp.float32),
            rows_v=pltpu.VMEM((NB, EB, o), jnp.float32),
            accum=pltpu.VMEM_SHARED((n, o), jnp.float32),
            isem=pltpu.SemaphoreType.DMA((NI,)),
            wsem=pltpu.SemaphoreType.DMA((NW,)),
            gsem=pltpu.SemaphoreType.DMA((NB,)),
            ssem=pltpu.SemaphoreType.DMA((NB,)),
        ),
    )
    def k(h_hbm, eib_hbm, w_hbm, out_hbm,
          idx_v, w_v, rows_v, accum, isem, wsem, gsem, ssem):
        c = lax.axis_index("c")
        s = lax.axis_index("s")
        blk0 = jnp.where(c == 0, s * k0, NS * k0 + s * k1)
        bpw = jnp.where(c == 0, k0, k1)

        r0 = s * rows_per_tile
        zb = rows_v.at[2]
        zv = jnp.zeros((LANES,), jnp.float32)
        for zr in range(EB):
            zrow = zb.at[zr]
            for zc in range(o // LANES):
                zrow[pl.ds(zc * LANES, LANES)] = zv
        nfull = rows_per_tile // EB
        for j in range(nfull):
            pltpu.sync_copy(zb, accum.at[pl.ds(r0 + j * EB, EB)])
        rem = rows_per_tile - nfull * EB
        if rem:
            pltpu.sync_copy(zb.at[pl.ds(0, rem)],
                            accum.at[pl.ds(r0 + nfull * EB, rem)])

        for j in range(3):
            pltpu.async_copy(eib_hbm.at[blk0 + j], idx_v.at[j], isem.at[j])
        for j in range(2):
            pltpu.async_copy(w_hbm.at[blk0 + j], w_v.at[j], wsem.at[j])
        plsc.subcore_barrier()
        for j in range(2):
            pltpu.make_async_copy(eib_hbm.at[blk0 + j], idx_v.at[j],
                                  isem.at[j]).wait()
            pltpu.async_copy(h_hbm.at[idx_v.at[j, 0]], rows_v.at[j],
                             gsem.at[j])

        def body(i, _):
            b = lax.rem(i, NB)
            si = lax.rem(i, NI)
            sw = lax.rem(i, NW)

            pltpu.make_async_copy(w_hbm.at[blk0 + i], w_v.at[sw],
                                  wsem.at[sw]).wait()
            pltpu.make_async_copy(h_hbm.at[idx_v.at[si, 0]], rows_v.at[b],
                                  gsem.at[b]).wait()

            buf = rows_v.at[b]
            for g in range(EB // LANES):
                w16 = w_v[sw, pl.ds(g * LANES, LANES)]
                for k in range(LANES):
                    sel = jnp.full((LANES,), k, jnp.int32)
                    wspl = w16.at[sel].get(mode="promise_in_bounds")
                    row = buf.at[g * LANES + k]
                    for cc in range(o // LANES):
                        sl = pl.ds(cc * LANES, LANES)
                        row[sl] = row[sl] * wspl

            pltpu.async_copy(rows_v.at[b], accum.at[idx_v.at[si, 1]],
                             ssem.at[b], add=True)

            @pl.when(i >= 1)
            def _():
                bq = lax.rem(i + 2, NB)
                sq = lax.rem(i + NI - 1, NI)
                pltpu.make_async_copy(rows_v.at[bq],
                                      accum.at[idx_v.at[sq, 1]],
                                      ssem.at[bq]).wait()

            @pl.when(i + 2 < bpw)
            def _():
                bg = lax.rem(i + 2, NB)
                sg = lax.rem(i + 2, NI)
                wg = lax.rem(i + 2, NW)
                pltpu.make_async_copy(eib_hbm.at[blk0 + i + 2],
                                      idx_v.at[sg], isem.at[sg]).wait()
                pltpu.async_copy(h_hbm.at[idx_v.at[sg, 0]], rows_v.at[bg],
                                 gsem.at[bg])
                pltpu.async_copy(w_hbm.at[blk0 + i + 2], w_v.at[wg],
                                 wsem.at[wg])

            @pl.when(i + 3 < bpw)
            def _():
                sn = lax.rem(i + 3, NI)
                pltpu.async_copy(eib_hbm.at[blk0 + i + 3], idx_v.at[sn],
                                 isem.at[sn])

            return 0

        lax.fori_loop(0, bpw, body, 0)

        last = bpw - 1
        pltpu.make_async_copy(rows_v.at[lax.rem(last, NB)],
                              accum.at[idx_v.at[lax.rem(last, NI), 1]],
                              ssem.at[lax.rem(last, NB)]).wait()
        plsc.subcore_barrier()

        pltpu.sync_copy(accum.at[pl.ds(r0, rows_per_tile)],
                        out_hbm.at[c, pl.ds(r0, rows_per_tile)])

    return k(h, eib, whb)


def kernel(x, edge_index, edge_weight, W):
    n, d = x.shape
    o = W.shape[1]
    e = edge_weight.shape[0]

    h = _matmul(x, W)

    nw = NC * NS
    bpw = -(-e // (nw * EB))
    k0 = (2 * bpw * 5) // 6
    k1 = 2 * bpw - k0
    ep = NS * (k0 + k1) * EB
    pad = ep - e
    src = jnp.concatenate([edge_index[0], jnp.zeros((pad,), jnp.int32)])
    dst = jnp.concatenate([edge_index[1], jnp.zeros((pad,), jnp.int32)])
    ew = jnp.concatenate([edge_weight, jnp.zeros((pad,), jnp.float32)])
    eib = jnp.stack(
        [src.reshape(ep // EB, EB), dst.reshape(ep // EB, EB)], axis=1)
    whb = ew.reshape(ep // EB, EB)

    n_pad = -(-n // (NS * 8)) * NS * 8
    partials = _sc_aggregate(h, eib, whb, k0, k1, n_pad, o)
    return _combine(partials[0, :n], partials[1, :n])


# speedup vs baseline: 1.0470x; 1.0470x over previous
import functools

import jax
import jax.numpy as jnp
from jax import lax
from jax.experimental import pallas as pl
from jax.experimental.pallas import tpu as pltpu
from jax.experimental.pallas import tpu_sc as plsc

NC = 2
NS = 16
LANES = 16
EB = 112


def _matmul_body(x_ref, w_ref, o_ref):
    o_ref[...] = jnp.dot(x_ref[...], w_ref[...],
                         preferred_element_type=jnp.float32)


def _matmul(x, W, block_rows=1000):
    n, d = x.shape
    o = W.shape[1]
    grid = n // block_rows
    return pl.pallas_call(
        _matmul_body,
        grid=(grid,),
        in_specs=[
            pl.BlockSpec((block_rows, d), lambda i: (i, 0)),
            pl.BlockSpec((d, o), lambda i: (0, 0)),
        ],
        out_specs=pl.BlockSpec((block_rows, o), lambda i: (i, 0)),
        out_shape=jax.ShapeDtypeStruct((n, o), jnp.float32),
    )(x, W)


def _combine_body(p_ref, o_ref):
    o_ref[...] = jnp.maximum(p_ref[0] + p_ref[1], 0.0)


def _combine(partials, n, block_rows=1000):
    o = partials.shape[-1]
    grid = n // block_rows
    return pl.pallas_call(
        _combine_body,
        grid=(grid,),
        in_specs=[pl.BlockSpec((NC, block_rows, o), lambda i: (0, i, 0))],
        out_specs=pl.BlockSpec((block_rows, o), lambda i: (i, 0)),
        out_shape=jax.ShapeDtypeStruct((n, o), jnp.float32),
    )(partials)


def _sc_aggregate(h, eib, whb, k0, k1, n, o):
    mesh = plsc.VectorSubcoreMesh(core_axis_name="c", subcore_axis_name="s")
    rows_per_tile = n // NS

    NB = 3
    NI = 4
    NW = 3

    @functools.partial(
        pl.kernel,
        out_type=jax.ShapeDtypeStruct((NC, n, o), jnp.float32),
        mesh=mesh,
        scratch_types=dict(
            idx_v=pltpu.VMEM((NI, 2, EB), jnp.int32),
            w_v=pltpu.VMEM((NW, EB), jnp.float32),
            rows_v=pltpu.VMEM((NB, EB, o), jnp.float32),
            accum=pltpu.VMEM_SHARED((n, o), jnp.float32),
            isem=pltpu.SemaphoreType.DMA((NI,)),
            wsem=pltpu.SemaphoreType.DMA((NW,)),
            gsem=pltpu.SemaphoreType.DMA((NB,)),
            ssem=pltpu.SemaphoreType.DMA((NB,)),
        ),
    )
    def k(h_hbm, eib_hbm, w_hbm, out_hbm,
          idx_v, w_v, rows_v, accum, isem, wsem, gsem, ssem):
        c = lax.axis_index("c")
        s = lax.axis_index("s")
        blk0 = jnp.where(c == 0, s * k0, NS * k0 + s * k1)
        bpw = jnp.where(c == 0, k0, k1)

        r0 = s * rows_per_tile
        zb = rows_v.at[2]
        zv = jnp.zeros((LANES,), jnp.float32)
        for zr in range(EB):
            zrow = zb.at[zr]
            for zc in range(o // LANES):
                zrow[pl.ds(zc * LANES, LANES)] = zv
        nfull = rows_per_tile // EB
        for j in range(nfull):
            pltpu.sync_copy(zb, accum.at[pl.ds(r0 + j * EB, EB)])
        rem = rows_per_tile - nfull * EB
        if rem:
            pltpu.sync_copy(zb.at[pl.ds(0, rem)],
                            accum.at[pl.ds(r0 + nfull * EB, rem)])

        for j in range(3):
            pltpu.async_copy(eib_hbm.at[blk0 + j], idx_v.at[j], isem.at[j])
        for j in range(2):
            pltpu.async_copy(w_hbm.at[blk0 + j], w_v.at[j], wsem.at[j])
        plsc.subcore_barrier()
        for j in range(2):
            pltpu.make_async_copy(eib_hbm.at[blk0 + j], idx_v.at[j],
                                  isem.at[j]).wait()
            pltpu.async_copy(h_hbm.at[idx_v.at[j, 0]], rows_v.at[j],
                             gsem.at[j])

        def body(i, _):
            b = lax.rem(i, NB)
            si = lax.rem(i, NI)
            sw = lax.rem(i, NW)

            pltpu.make_async_copy(w_hbm.at[blk0 + i], w_v.at[sw],
                                  wsem.at[sw]).wait()
            pltpu.make_async_copy(h_hbm.at[idx_v.at[si, 0]], rows_v.at[b],
                                  gsem.at[b]).wait()

            buf = rows_v.at[b]
            for g in range(EB // LANES):
                w16 = w_v[sw, pl.ds(g * LANES, LANES)]
                for k in range(LANES):
                    sel = jnp.full((LANES,), k, jnp.int32)
                    wspl = w16.at[sel].get(mode="promise_in_bounds")
                    row = buf.at[g * LANES + k]
                    for cc in range(o // LANES):
                        sl = pl.ds(cc * LANES, LANES)
                        row[sl] = row[sl] * wspl

            pltpu.async_copy(rows_v.at[b], accum.at[idx_v.at[si, 1]],
                             ssem.at[b], add=True)

            @pl.when(i >= 1)
            def _():
                bq = lax.rem(i + 2, NB)
                sq = lax.rem(i + NI - 1, NI)
                pltpu.make_async_copy(rows_v.at[bq],
                                      accum.at[idx_v.at[sq, 1]],
                                      ssem.at[bq]).wait()

            @pl.when(i + 2 < bpw)
            def _():
                bg = lax.rem(i + 2, NB)
                sg = lax.rem(i + 2, NI)
                wg = lax.rem(i + 2, NW)
                pltpu.make_async_copy(eib_hbm.at[blk0 + i + 2],
                                      idx_v.at[sg], isem.at[sg]).wait()
                pltpu.async_copy(h_hbm.at[idx_v.at[sg, 0]], rows_v.at[bg],
                                 gsem.at[bg])
                pltpu.async_copy(w_hbm.at[blk0 + i + 2], w_v.at[wg],
                                 wsem.at[wg])

            @pl.when(i + 3 < bpw)
            def _():
                sn = lax.rem(i + 3, NI)
                pltpu.async_copy(eib_hbm.at[blk0 + i + 3], idx_v.at[sn],
                                 isem.at[sn])

            return 0

        lax.fori_loop(0, bpw, body, 0)

        last = bpw - 1
        pltpu.make_async_copy(rows_v.at[lax.rem(last, NB)],
                              accum.at[idx_v.at[lax.rem(last, NI), 1]],
                              ssem.at[lax.rem(last, NB)]).wait()
        plsc.subcore_barrier()

        pltpu.sync_copy(accum.at[pl.ds(r0, rows_per_tile)],
                        out_hbm.at[c, pl.ds(r0, rows_per_tile)])

    return k(h, eib, whb)


def kernel(x, edge_index, edge_weight, W):
    n, d = x.shape
    o = W.shape[1]
    e = edge_weight.shape[0]

    h = _matmul(x, W)

    nw = NC * NS
    bpw = -(-e // (nw * EB))
    k0 = (2 * bpw * 4) // 5
    k1 = 2 * bpw - k0
    ep = NS * (k0 + k1) * EB
    pad = ep - e
    src = jnp.concatenate([edge_index[0], jnp.zeros((pad,), jnp.int32)])
    dst = jnp.concatenate([edge_index[1], jnp.zeros((pad,), jnp.int32)])
    ew = jnp.concatenate([edge_weight, jnp.zeros((pad,), jnp.float32)])
    eib = jnp.stack(
        [src.reshape(ep // EB, EB), dst.reshape(ep // EB, EB)], axis=1)
    whb = ew.reshape(ep // EB, EB)

    n_pad = -(-n // (NS * 8)) * NS * 8
    partials = _sc_aggregate(h, eib, whb, k0, k1, n_pad, o)
    return _combine(partials, n)
